# Initial kernel scaffold; baseline (speedup 1.0000x reference)
import jax
import jax.numpy as jnp
from jax.experimental import pallas as pl


def kernel(centers, sigmas, intensities):
    raise NotImplementedError("write your pallas kernel here")


# speedup vs baseline: 2.3772x; 2.3772x over previous
import jax
import jax.numpy as jnp
from jax.experimental import pallas as pl
from jax.experimental.pallas import tpu as pltpu

N = 512
D = 128
JB = 8
SF = float(D - 1)


def _factors(ci, si, axis_first):
    if axis_first:
        ii = jax.lax.broadcasted_iota(jnp.int32, (D, N), 0).astype(jnp.float32)
        c = ci.reshape(1, N)
        s = si.reshape(1, N)
    else:
        ii = jax.lax.broadcasted_iota(jnp.int32, (N, D), 1).astype(jnp.float32)
        c = ci.reshape(N, 1)
        s = si.reshape(N, 1)
    c_idx = c * SF
    cut = 3.0 * s * SF
    lo = jnp.floor(jnp.maximum(c_idx - cut, 0.0))
    hi = jnp.minimum(jnp.floor(jnp.minimum(c_idx + cut, SF) + 1.0), float(D))
    mask = (ii >= lo) & (ii < hi)
    coords = ii * jnp.float32(1.0 / SF)
    g = jnp.exp(-0.5 * (coords - c) ** 2 / (s * s))
    return jnp.where(mask, g, 0.0)


def _body(params_ref, out_ref, gxT_ref, gyT_ref, gz_ref):
    i = pl.program_id(0)

    @pl.when(i == 0)
    def _():
        cx = params_ref[0, :]
        cy = params_ref[1, :]
        cz = params_ref[2, :]
        sg = params_ref[3, :]
        inten = params_ref[4, :]
        gxT_ref[...] = _factors(cx, sg, True) * inten.reshape(1, N)
        gyT_ref[...] = _factors(cy, sg, True)
        gz_ref[...] = _factors(cz, sg, False)

    xb = gxT_ref[pl.ds(i * JB, JB), :]
    B = (xb[:, None, :] * gyT_ref[...][None, :, :]).reshape(JB * D, N)
    O = jnp.dot(B, gz_ref[...], preferred_element_type=jnp.float32)
    out_ref[...] = O.reshape(JB, D, D)


def kernel(centers, sigmas, intensities):
    params = jnp.zeros((8, N), jnp.float32)
    params = params.at[0].set(centers[:, 0])
    params = params.at[1].set(centers[:, 1])
    params = params.at[2].set(centers[:, 2])
    params = params.at[3].set(sigmas)
    params = params.at[4].set(intensities)

    out = pl.pallas_call(
        _body,
        grid=(D // JB,),
        in_specs=[pl.BlockSpec((8, N), lambda j: (0, 0))],
        out_specs=pl.BlockSpec((JB, D, D), lambda i: (i, 0, 0)),
        out_shape=jax.ShapeDtypeStruct((D, D, D), jnp.float32),
        scratch_shapes=[
            pltpu.VMEM((D, N), jnp.float32),
            pltpu.VMEM((D, N), jnp.float32),
            pltpu.VMEM((N, D), jnp.float32),
        ],
    )(params)
    return out
